# baseline (speedup 1.0000x reference)
import jax
import jax.numpy as jnp
from jax.experimental import pallas as pl
from jax.experimental.pallas import tpu as pltpu

_B = 128
_V = 100000
_CH = 10000
_NC = _V // _CH


def _softmax_kernel(x_hbm, g_ref, o_hbm, xbuf, sin, sout):
    def in_copy(k):
        return pltpu.make_async_copy(
            x_hbm.at[pl.ds(k * _CH, _CH)],
            xbuf.at[pl.ds(pl.multiple_of(k * _CH, 8), _CH)],
            sin.at[k],
        )

    def out_copy(k):
        return pltpu.make_async_copy(
            xbuf.at[pl.ds(pl.multiple_of(k * _CH, 8), _CH)],
            o_hbm.at[pl.ds(k * _CH, _CH)],
            sout.at[k],
        )

    for k in range(_NC):
        in_copy(k).start()

    gold = g_ref[...]
    vval = jnp.float32(_V)
    iota = jax.lax.broadcasted_iota(jnp.int32, (_CH, _B), 0)

    def masked(k):
        x = xbuf[pl.ds(pl.multiple_of(k * _CH, 8), _CH), :]
        rowid = iota + k * _CH
        return jnp.where(rowid == gold, vval, x)

    def step_a(k, carry):
        m_old, s_old = carry
        in_copy(k).wait()
        y = masked(k)
        m_c = jnp.max(y, axis=0, keepdims=True)
        s_c = jnp.sum(jnp.exp(y - m_c), axis=0, keepdims=True)
        m_new = jnp.maximum(m_old, m_c)
        s_new = s_old * jnp.exp(m_old - m_new) + s_c * jnp.exp(m_c - m_new)
        return m_new, s_new

    m0 = jnp.full((1, _B), -jnp.inf, jnp.float32)
    s0 = jnp.zeros((1, _B), jnp.float32)
    m, s = jax.lax.fori_loop(0, _NC, step_a, (m0, s0))
    c = m + jnp.log(s)

    def step_b(k, _):
        y = masked(k)
        xbuf[pl.ds(pl.multiple_of(k * _CH, 8), _CH), :] = jnp.exp(y - c)
        out_copy(k).start()
        return 0

    jax.lax.fori_loop(0, _NC, step_b, 0)
    for k in range(_NC):
        out_copy(k).wait()


def kernel(t, gold):
    tt = t.T
    g2 = gold.reshape(1, _B)

    out_t = pl.pallas_call(
        _softmax_kernel,
        grid=(1,),
        in_specs=[
            pl.BlockSpec(memory_space=pl.ANY),
            pl.BlockSpec((1, _B), lambda i: (0, 0)),
        ],
        out_specs=pl.BlockSpec(memory_space=pl.ANY),
        out_shape=jax.ShapeDtypeStruct((_V, _B), jnp.float32),
        scratch_shapes=[
            pltpu.VMEM((_V, _B), jnp.float32),
            pltpu.SemaphoreType.DMA((_NC,)),
            pltpu.SemaphoreType.DMA((_NC,)),
        ],
        compiler_params=pltpu.CompilerParams(
            vmem_limit_bytes=100 * 1024 * 1024,
        ),
    )(tt, g2)

    return out_t.T


# speedup vs baseline: 1.5502x; 1.5502x over previous
import jax
import jax.numpy as jnp
from jax.experimental import pallas as pl
from jax.experimental.pallas import tpu as pltpu

_B = 128
_V = 100000
_CH = 5000
_NC = _V // _CH


def _softmax_kernel(x_hbm, g_ref, o_hbm, xbuf, fvm, fsm, sin, sout, sflag):
    def in_copy(k):
        return pltpu.make_async_copy(
            x_hbm.at[pl.ds(k * _CH, _CH)],
            xbuf.at[pl.ds(pl.multiple_of(k * _CH, 8), _CH)],
            sin.at[k],
        )

    def out_copy(k):
        return pltpu.make_async_copy(
            xbuf.at[pl.ds(pl.multiple_of(k * _CH, 8), _CH)],
            o_hbm.at[pl.ds(k * _CH, _CH)],
            sout.at[k],
        )

    for k in range(_NC):
        in_copy(k).start()

    gold = g_ref[...]
    vval = jnp.float32(_V)
    iota = jax.lax.broadcasted_iota(jnp.int32, (_CH, _B), 0)

    def masked(k):
        x = xbuf[pl.ds(pl.multiple_of(k * _CH, 8), _CH), :]
        rowid = iota + k * _CH
        return jnp.where(rowid == gold, vval, x)

    def step_a(k, carry):
        m_old, s_old = carry
        in_copy(k).wait()
        y = masked(k)
        m_c = jnp.max(y, axis=0, keepdims=True)
        s_c = jnp.sum(jnp.exp(y - m_c), axis=0, keepdims=True)
        m_new = jnp.maximum(m_old, m_c)
        s_new = s_old * jnp.exp(m_old - m_new) + s_c * jnp.exp(m_c - m_new)
        xbuf[pl.ds(pl.multiple_of(k * _CH, 8), _CH), :] = jnp.exp(y - vval)
        out_copy(k).start()
        return m_new, s_new

    m0 = jnp.full((1, _B), -jnp.inf, jnp.float32)
    s0 = jnp.zeros((1, _B), jnp.float32)
    m, s = jax.lax.fori_loop(0, _NC, step_a, (m0, s0))
    c = m + jnp.log(s)

    nbad = jnp.sum((c != vval).astype(jnp.int32), axis=1, keepdims=True)
    fvm[...] = nbad
    pltpu.make_async_copy(fvm, fsm, sflag).start()
    pltpu.make_async_copy(fvm, fsm, sflag).wait()

    for k in range(_NC):
        out_copy(k).wait()

    @pl.when(fsm[0, 0] != 0)
    def _():
        for k2 in range(_NC):
            in_copy(k2).start()

        def step_r(k, _):
            in_copy(k).wait()
            y = masked(k)
            xbuf[pl.ds(pl.multiple_of(k * _CH, 8), _CH), :] = jnp.exp(y - c)
            out_copy(k).start()
            return 0

        jax.lax.fori_loop(0, _NC, step_r, 0)

        def wait_r(k, _):
            out_copy(k).wait()
            return 0

        jax.lax.fori_loop(0, _NC, wait_r, 0)


def kernel(t, gold):
    tt = t.T
    g2 = gold.reshape(1, _B)

    out_t = pl.pallas_call(
        _softmax_kernel,
        grid=(1,),
        in_specs=[
            pl.BlockSpec(memory_space=pl.ANY),
            pl.BlockSpec((1, _B), lambda i: (0, 0)),
        ],
        out_specs=pl.BlockSpec(memory_space=pl.ANY),
        out_shape=jax.ShapeDtypeStruct((_V, _B), jnp.float32),
        scratch_shapes=[
            pltpu.VMEM((_V, _B), jnp.float32),
            pltpu.VMEM((1, 1), jnp.int32),
            pltpu.SMEM((1, 1), jnp.int32),
            pltpu.SemaphoreType.DMA((_NC,)),
            pltpu.SemaphoreType.DMA((_NC,)),
            pltpu.SemaphoreType.DMA,
        ],
        compiler_params=pltpu.CompilerParams(
            vmem_limit_bytes=100 * 1024 * 1024,
        ),
    )(tt, g2)

    return out_t.T
